# baseline (speedup 1.0000x reference)
import jax
import jax.numpy as jnp
from jax import lax
from jax.experimental import pallas as pl
from jax.experimental.pallas import tpu as pltpu
from jax.experimental.pallas import tpu_sc as plsc

VOCAB = 100000
EMB = 64
CTX = 20
HID = 128
NIDX = 2 * CTX
FLAT = NIDX * EMB
WIDE = 2 * EMB
NWROWS = VOCAB // 2

BC = 8192
NB = -(-VOCAB // BC)
PADV = NB * BC

IDX_PAD = 48


def _sc_gather_body(table_hbm, idx_hbm, out_hbm, idx_v, shift_v, rows_v, sem):
    wid = lax.axis_index("s") * 2 + lax.axis_index("c")

    @pl.when(wid == 0)
    def _():
        pltpu.sync_copy(idx_hbm, idx_v.at[pl.ds(0, NIDX)])
        for c in range(IDX_PAD // 16):
            v = idx_v[pl.ds(c * 16, 16)]
            s = lax.shift_right_logical(v, 1)
            s = jnp.minimum(jnp.maximum(s, 0), NWROWS - 1)
            shift_v[pl.ds(c * 16, 16)] = s
        pltpu.async_copy(table_hbm.at[shift_v], rows_v, sem).wait()
        pltpu.sync_copy(rows_v.at[pl.ds(0, NIDX)], out_hbm)


def _sc_gather(table_wide, idx):
    mesh = plsc.VectorSubcoreMesh(core_axis_name="c", subcore_axis_name="s")
    k = pl.kernel(
        _sc_gather_body,
        out_type=jax.ShapeDtypeStruct((NIDX, WIDE), jnp.float32),
        mesh=mesh,
        scratch_types=[
            pltpu.VMEM((IDX_PAD,), jnp.int32),
            pltpu.VMEM((IDX_PAD,), jnp.int32),
            pltpu.VMEM((IDX_PAD, WIDE), jnp.float32),
            pltpu.SemaphoreType.DMA,
        ],
    )
    return k(table_wide, idx)


def _tc_body(xl_ref, xr_ref, pf_ref, w1_ref, b1_ref, w2_ref, b2_ref, out_ref,
             hid_ref, m_ref, s_ref):
    j = pl.program_id(0)

    @pl.when(j == 0)
    def _init():
        pf = pf_ref[...]
        x = xl_ref[...] + pf * (xr_ref[...] - xl_ref[...])
        h = jnp.dot(x, w1_ref[...], preferred_element_type=jnp.float32)
        hid_ref[...] = jnp.maximum(h + b1_ref[...], 0.0)
        m_ref[0] = -jnp.inf
        s_ref[0] = 0.0

    blk = jnp.dot(hid_ref[...], w2_ref[...], preferred_element_type=jnp.float32)
    blk = blk + b2_ref[...]
    col = j * BC + lax.broadcasted_iota(jnp.int32, (1, BC), 1)
    valid = col < VOCAB
    bm = jnp.max(jnp.where(valid, blk, -jnp.inf))
    m_old = m_ref[0]
    m_new = jnp.maximum(m_old, bm)
    s_ref[0] = (s_ref[0] * jnp.exp(m_old - m_new)
                + jnp.sum(jnp.where(valid, jnp.exp(blk - m_new), 0.0)))
    m_ref[0] = m_new

    off = pl.multiple_of(j * BC, BC)
    out_ref[:, pl.ds(off, BC)] = blk

    @pl.when(j == NB - 1)
    def _fin():
        c = m_ref[0] + jnp.log(s_ref[0])
        out_ref[...] = out_ref[...] - c


def _tc_mlp(xl, xr, pf, W1, b1, W2, b2):
    out = pl.pallas_call(
        _tc_body,
        grid=(NB,),
        in_specs=[
            pl.BlockSpec((1, FLAT), lambda j: (0, 0)),
            pl.BlockSpec((1, FLAT), lambda j: (0, 0)),
            pl.BlockSpec((1, FLAT), lambda j: (0, 0)),
            pl.BlockSpec((FLAT, HID), lambda j: (0, 0)),
            pl.BlockSpec((1, HID), lambda j: (0, 0)),
            pl.BlockSpec((HID, BC), lambda j: (0, j)),
            pl.BlockSpec((1, BC), lambda j: (0, j)),
        ],
        out_specs=pl.BlockSpec((1, PADV), lambda j: (0, 0)),
        out_shape=jax.ShapeDtypeStruct((1, PADV), jnp.float32),
        scratch_shapes=[
            pltpu.VMEM((1, HID), jnp.float32),
            pltpu.SMEM((1,), jnp.float32),
            pltpu.SMEM((1,), jnp.float32),
        ],
    )(xl, xr, pf, W1, b1, W2, b2)
    return out[:, :VOCAB]


def kernel(inputs, table, W1, b1, W2, b2):
    wide_rows = _sc_gather(table.reshape(NWROWS, WIDE), inputs)
    xl = wide_rows[:, :EMB].reshape(1, FLAT)
    xr = wide_rows[:, EMB:].reshape(1, FLAT)
    pf = jnp.repeat((inputs & 1).astype(jnp.float32), EMB).reshape(1, FLAT)
    return _tc_mlp(xl, xr, pf, W1, b1.reshape(1, HID), W2,
                   b2.reshape(1, VOCAB))


# speedup vs baseline: 1.2572x; 1.2572x over previous
import jax
import jax.numpy as jnp
from jax import lax
from jax.experimental import pallas as pl
from jax.experimental.pallas import tpu as pltpu
from jax.experimental.pallas import tpu_sc as plsc

VOCAB = 100000
EMB = 64
CTX = 20
HID = 128
NIDX = 2 * CTX
FLAT = NIDX * EMB

BC = 8192
NB = -(-VOCAB // BC)
PADV = NB * BC

IDX_PAD = 48


def _sc_gather_body(table_hbm, idx_hbm, out_hbm, idx_v, rows_v, sem):
    wid = lax.axis_index("s") * 2 + lax.axis_index("c")

    @pl.when(wid == 0)
    def _():
        pltpu.sync_copy(idx_hbm, idx_v.at[pl.ds(0, NIDX)])
        copies = []
        for i in range(NIDX):
            c, l = divmod(i, 16)
            v = idx_v[pl.ds(c * 16, 16)]
            s = v[l]
            s = jnp.minimum(jnp.maximum(s, 0), VOCAB - 1)
            copies.append(pltpu.async_copy(
                table_hbm.at[pl.ds(s, 1)], rows_v.at[pl.ds(i, 1)], sem))
        for cp in copies:
            cp.wait()
        pltpu.sync_copy(rows_v, out_hbm)


def _sc_gather(table, idx):
    mesh = plsc.VectorSubcoreMesh(core_axis_name="c", subcore_axis_name="s")
    k = pl.kernel(
        _sc_gather_body,
        out_type=jax.ShapeDtypeStruct((NIDX, EMB), jnp.float32),
        mesh=mesh,
        scratch_types=[
            pltpu.VMEM((IDX_PAD,), jnp.int32),
            pltpu.VMEM((NIDX, EMB), jnp.float32),
            pltpu.SemaphoreType.DMA,
        ],
    )
    return k(table, idx)


def _tc_body(emb_ref, w1_ref, b1_ref, w2_ref, b2_ref, out_ref,
             hid_ref, m_ref, s_ref):
    j = pl.program_id(0)

    @pl.when(j == 0)
    def _init():
        h = b1_ref[...]
        for i in range(NIDX):
            h = h + jnp.dot(emb_ref[pl.ds(i, 1), :], w1_ref[i],
                            preferred_element_type=jnp.float32)
        hid_ref[...] = jnp.maximum(h, 0.0)
        m_ref[0] = -jnp.inf
        s_ref[0] = 0.0

    blk = jnp.dot(hid_ref[...], w2_ref[...], preferred_element_type=jnp.float32)
    blk = blk + b2_ref[...]
    col = j * BC + lax.broadcasted_iota(jnp.int32, (1, BC), 1)
    valid = col < VOCAB
    bm = jnp.max(jnp.where(valid, blk, -jnp.inf))
    m_old = m_ref[0]
    m_new = jnp.maximum(m_old, bm)
    s_ref[0] = (s_ref[0] * jnp.exp(m_old - m_new)
                + jnp.sum(jnp.where(valid, jnp.exp(blk - m_new), 0.0)))
    m_ref[0] = m_new

    off = pl.multiple_of(j * BC, BC)
    out_ref[:, pl.ds(off, BC)] = blk

    @pl.when(j == NB - 1)
    def _fin():
        c = m_ref[0] + jnp.log(s_ref[0])
        out_ref[...] = out_ref[...] - c


def _tc_mlp(emb, W1r, b1, W2, b2):
    out = pl.pallas_call(
        _tc_body,
        grid=(NB,),
        in_specs=[
            pl.BlockSpec((NIDX, EMB), lambda j: (0, 0)),
            pl.BlockSpec((NIDX, EMB, HID), lambda j: (0, 0, 0)),
            pl.BlockSpec((1, HID), lambda j: (0, 0)),
            pl.BlockSpec((HID, BC), lambda j: (0, j)),
            pl.BlockSpec((1, BC), lambda j: (0, j)),
        ],
        out_specs=pl.BlockSpec((1, PADV), lambda j: (0, 0)),
        out_shape=jax.ShapeDtypeStruct((1, PADV), jnp.float32),
        scratch_shapes=[
            pltpu.VMEM((1, HID), jnp.float32),
            pltpu.SMEM((1,), jnp.float32),
            pltpu.SMEM((1,), jnp.float32),
        ],
    )(emb, W1r, b1, W2, b2)
    return out[:, :VOCAB]


def kernel(inputs, table, W1, b1, W2, b2):
    emb = _sc_gather(table, inputs)
    W1r = W1.reshape(NIDX, EMB, HID)
    return _tc_mlp(emb, W1r, b1.reshape(1, HID), W2, b2.reshape(1, VOCAB))


# speedup vs baseline: 1.2913x; 1.0271x over previous
import jax
import jax.numpy as jnp
from jax import lax
from jax.experimental import pallas as pl
from jax.experimental.pallas import tpu as pltpu
from jax.experimental.pallas import tpu_sc as plsc

VOCAB = 100000
EMB = 64
CTX = 20
HID = 128
NIDX = 2 * CTX
FLAT = NIDX * EMB

BC = 16384
NSPLIT = 1
NBH = -(-VOCAB // (BC * NSPLIT))
PADV = NSPLIT * NBH * BC

IDX_PAD = 48


def _sc_gather_body(table_hbm, idx_hbm, out_hbm, idx_v, rows_v, sem):
    wid = lax.axis_index("s") * 2 + lax.axis_index("c")

    @pl.when(wid == 0)
    def _():
        pltpu.sync_copy(idx_hbm, idx_v.at[pl.ds(0, NIDX)])
        copies = []
        for i in range(NIDX):
            c, l = divmod(i, 16)
            v = idx_v[pl.ds(c * 16, 16)]
            s = v[l]
            s = jnp.minimum(jnp.maximum(s, 0), VOCAB - 1)
            copies.append(pltpu.async_copy(
                table_hbm.at[pl.ds(s, 1)], rows_v.at[pl.ds(i, 1)], sem))
        for cp in copies:
            cp.wait()
        pltpu.sync_copy(rows_v, out_hbm)


def _sc_gather(table, idx):
    mesh = plsc.VectorSubcoreMesh(core_axis_name="c", subcore_axis_name="s")
    k = pl.kernel(
        _sc_gather_body,
        out_type=jax.ShapeDtypeStruct((NIDX, EMB), jnp.float32),
        mesh=mesh,
        scratch_types=[
            pltpu.VMEM((IDX_PAD,), jnp.int32),
            pltpu.VMEM((NIDX, EMB), jnp.float32),
            pltpu.SemaphoreType.DMA,
        ],
    )
    return k(table, idx)


def _tc_body(*refs):
    emb_ref, w1_ref, b1_ref = refs[0], refs[1], refs[2]
    w2_refs = refs[3:3 + NSPLIT]
    b2_refs = refs[3 + NSPLIT:3 + 2 * NSPLIT]
    out_ref = refs[3 + 2 * NSPLIT]
    hid_ref, m_ref, s_ref = refs[4 + 2 * NSPLIT:]
    j = pl.program_id(0)

    @pl.when(j == 0)
    def _init():
        h = b1_ref[...]
        for i in range(NIDX):
            h = h + jnp.dot(emb_ref[pl.ds(i, 1), :], w1_ref[i],
                            preferred_element_type=jnp.float32)
        hid_ref[...] = jnp.maximum(h, 0.0)
        m_ref[0] = -jnp.inf
        s_ref[0] = 0.0

    hid = hid_ref[...]
    iota = lax.broadcasted_iota(jnp.int32, (1, BC), 1)
    blks, valids = [], []
    bm = None
    for k in range(NSPLIT):
        blk = jnp.dot(hid, w2_refs[k][...], preferred_element_type=jnp.float32)
        blk = blk + b2_refs[k][...]
        col = (k * NBH + j) * BC + iota
        valid = col < VOCAB
        m_k = jnp.max(jnp.where(valid, blk, -jnp.inf))
        bm = m_k if bm is None else jnp.maximum(bm, m_k)
        blks.append(blk)
        valids.append(valid)
    m_old = m_ref[0]
    m_new = jnp.maximum(m_old, bm)
    s_add = 0.0
    for k in range(NSPLIT):
        s_add = s_add + jnp.sum(
            jnp.where(valids[k], jnp.exp(blks[k] - m_new), 0.0))
    s_ref[0] = s_ref[0] * jnp.exp(m_old - m_new) + s_add
    m_ref[0] = m_new

    for k in range(NSPLIT):
        off = pl.multiple_of((k * NBH + j) * BC, BC)
        out_ref[:, pl.ds(off, BC)] = blks[k]

    @pl.when(j == NBH - 1)
    def _fin():
        c = m_ref[0] + jnp.log(s_ref[0])
        out_ref[...] = out_ref[...] - c


def _tc_mlp(emb, W1r, b1, W2, b2):
    w2_specs = [
        pl.BlockSpec((HID, BC), lambda j, k=k: (0, k * NBH + j))
        for k in range(NSPLIT)
    ]
    b2_specs = [
        pl.BlockSpec((1, BC), lambda j, k=k: (0, k * NBH + j))
        for k in range(NSPLIT)
    ]
    out = pl.pallas_call(
        _tc_body,
        grid=(NBH,),
        in_specs=[
            pl.BlockSpec((NIDX, EMB), lambda j: (0, 0)),
            pl.BlockSpec((NIDX, EMB, HID), lambda j: (0, 0, 0)),
            pl.BlockSpec((1, HID), lambda j: (0, 0)),
        ] + w2_specs + b2_specs,
        out_specs=pl.BlockSpec((1, PADV), lambda j: (0, 0)),
        out_shape=jax.ShapeDtypeStruct((1, PADV), jnp.float32),
        scratch_shapes=[
            pltpu.VMEM((1, HID), jnp.float32),
            pltpu.SMEM((1,), jnp.float32),
            pltpu.SMEM((1,), jnp.float32),
        ],
    )(emb, W1r, b1, *([W2] * NSPLIT), *([b2] * NSPLIT))
    return out[:, :VOCAB]


def kernel(inputs, table, W1, b1, W2, b2):
    emb = _sc_gather(table, inputs)
    W1r = W1.reshape(NIDX, EMB, HID)
    return _tc_mlp(emb, W1r, b1.reshape(1, HID), W2, b2.reshape(1, VOCAB))
